# baseline (speedup 1.0000x reference)
import jax
import jax.numpy as jnp
from jax.experimental import pallas as pl
from jax.experimental.pallas import tpu as pltpu

S = 2048
HID = 128
BSZ = 4
STACK = BSZ * HID
TIME_STEPS = 10
TEMPORAL_STEPS = 3
NSPLIT = 4
RSPLIT = S // NSPLIT
NTRANS = 2 * NSPLIT
NSTAGE = 3


def _swish(v):
    return v * jax.nn.sigmoid(v)


def _dot(a, b, dims):
    return jax.lax.dot_general(
        a, b, (dims, ((), ())), preferred_element_type=jnp.float32)


def _blockmix(w, xs):
    return jnp.concatenate(
        [_dot(w, xs[b * HID:(b + 1) * HID], ((0,), (0,))) for b in range(BSZ)],
        axis=0)


def _col(vec_ref):
    return jnp.transpose(jnp.reshape(vec_ref[...], (1, -1)))


def _col4(vec_ref):
    c = _col(vec_ref)
    return jnp.concatenate([c] * BSZ, axis=0)


def _fused_kernel(x0t_ref, b1_hbm, b2_hbm,
                  w_enc0_ref, b_enc0_ref, w2f_ref, b_enc2_ref,
                  w_c0_ref, w_c2_ref, alpha_ref,
                  w_f_ref, b_f_ref,
                  out_ref, stage, b2_s, sems):
    def _issue(i):
        src = b1_hbm if i < NSPLIT else b2_hbm
        c = i % NSPLIT
        pltpu.make_async_copy(src.at[pl.ds(c * RSPLIT, RSPLIT)],
                              stage.at[i % NSTAGE],
                              sems.at[i]).start()

    def _wait(i):
        src = b1_hbm if i < NSPLIT else b2_hbm
        c = i % NSPLIT
        pltpu.make_async_copy(src.at[pl.ds(c * RSPLIT, RSPLIT)],
                              stage.at[i % NSTAGE],
                              sems.at[i]).wait()

    _issue(0)
    _issue(1)

    alpha = alpha_ref[0]
    w_enc0 = w_enc0_ref[...]
    x0h = _swish(jnp.concatenate(
        [_dot(w_enc0, x0t_ref[:, b * S:(b + 1) * S], ((0,), (0,)))
         for b in range(BSZ)], axis=0) + _col4(b_enc0_ref))

    w_f = w_f_ref[...]
    b_f = _col(b_f_ref)
    w_c0 = w_c0_ref[...]

    fillers = (
        [lambda b=b: ('l0', b, _dot(w_c0, x0h[b * HID:(b + 1) * HID],
                                    ((0,), (0,)))) for b in range(BSZ)]
        + [lambda b=b: ('k0', b, _dot(w_f[0:HID], x0h[b * HID:(b + 1) * HID],
                                      ((0,), (0,)))) for b in range(BSZ)]
    )
    l0_1 = [None] * BSZ
    acc = [None] * BSZ

    def _fill(i):
        if i < len(fillers):
            tag, b, v = fillers[i]()
            if tag == 'l0':
                l0_1[b] = v
            else:
                acc[b] = v

    t1 = None
    for i in range(NSPLIT):
        _fill(i)
        _wait(i)
        if i + 2 < NTRANS:
            _issue(i + 2)
        p = _dot(x0h[:, i * RSPLIT:(i + 1) * RSPLIT],
                 stage[i % NSTAGE], ((1,), (0,)))
        t1 = p if t1 is None else t1 + p

    t2 = None
    for i in range(NSPLIT, NTRANS):
        c = i % NSPLIT
        _fill(i)
        _wait(i)
        if i + 2 < NTRANS:
            _issue(i + 2)
        chunk = stage[i % NSTAGE]
        b2_s[c * RSPLIT:(c + 1) * RSPLIT, :] = chunk
        p = _dot(t1[:, c * RSPLIT:(c + 1) * RSPLIT], chunk, ((1,), (0,)))
        t2 = p if t2 is None else t2 + p
    x2h = _blockmix(w2f_ref[...], t2) + _col4(b_enc2_ref)

    B2 = b2_s[...]
    w_c2 = w_c2_ref[...]

    for b in range(BSZ):
        sl = slice(b * HID, (b + 1) * HID)
        x2h_b = x2h[sl]
        up0 = _dot(x2h_b, B2, ((1,), (0,)))
        x0h_1 = _swish(up0 + alpha * (l0_1[b] - up0))
        x2h_1 = _swish(_dot(w_c2, x2h_b, ((0,), (0,))))
        a = acc[b] + _dot(w_f[HID:2 * HID], x0h_1, ((0,), (0,)))
        l0_2 = _dot(w_c0, x0h_1, ((0,), (0,)))
        up0b = _dot(x2h_1, B2, ((1,), (0,)))
        x0h_2 = _swish(up0b + alpha * (l0_2 - up0b))
        a = a + _dot(w_f[2 * HID:3 * HID], x0h_2, ((0,), (0,)))
        out_ref[b] = _swish(a + b_f)


def kernel(x, pos, batch, triangles, B1, B2, W_enc0, b_enc0, theta_edge,
           theta_tri, W_enc1, b_enc1, W_enc2, b_enc2, W_conv0, W_conv1,
           W_conv2, alpha, W_tproj, b_tproj, W_dec, b_dec):
    vfull = lambda shp: pl.BlockSpec(shp, lambda: (0,) * len(shp))
    hbm = pl.BlockSpec(memory_space=pl.ANY)
    smem1 = pl.BlockSpec(memory_space=pltpu.SMEM)

    x0t = jnp.concatenate([x.T, pos.T], axis=0)
    hp = 'highest'
    w2f = jnp.dot(jnp.dot(theta_edge, theta_tri, precision=hp), W_enc2,
                  precision=hp)
    w_f = jnp.dot(W_tproj, W_dec, precision=hp)
    b_f = jnp.dot(b_tproj, W_dec, precision=hp) + b_dec

    out = pl.pallas_call(
        _fused_kernel,
        in_specs=[
            vfull((5, BSZ * S)),
            hbm, hbm,
            vfull((5, HID)), vfull((HID,)),
            vfull((HID, HID)), vfull((HID,)),
            vfull((HID, HID)), vfull((HID, HID)),
            smem1,
            vfull((HID * TEMPORAL_STEPS, TIME_STEPS)), vfull((TIME_STEPS,)),
        ],
        out_specs=vfull((BSZ, TIME_STEPS, S)),
        out_shape=jax.ShapeDtypeStruct((BSZ, TIME_STEPS, S), jnp.float32),
        scratch_shapes=[
            pltpu.VMEM((NSTAGE, RSPLIT, S), jnp.float32),
            pltpu.VMEM((S, S), jnp.float32),
            pltpu.SemaphoreType.DMA((NTRANS,)),
        ],
        compiler_params=pltpu.CompilerParams(
            vmem_limit_bytes=110 * 1024 * 1024),
    )(
        x0t, B1, B2,
        W_enc0, b_enc0, w2f, b_enc2,
        W_conv0, W_conv2, alpha.reshape(1),
        w_f, b_f,
    )
    return out


# speedup vs baseline: 1.0092x; 1.0092x over previous
import jax
import jax.numpy as jnp
from jax.experimental import pallas as pl
from jax.experimental.pallas import tpu as pltpu

S = 2048
HID = 128
BSZ = 4
STACK = BSZ * HID
TIME_STEPS = 10
TEMPORAL_STEPS = 3
NSPLIT = 2
RSPLIT = S // NSPLIT
NSTAGE = 4


def _swish(v):
    return v * jax.nn.sigmoid(v)


def _dot(a, b, dims):
    return jax.lax.dot_general(
        a, b, (dims, ((), ())), preferred_element_type=jnp.float32)


def _blockmix(w, xs):
    return jnp.concatenate(
        [_dot(w, xs[b * HID:(b + 1) * HID], ((0,), (0,))) for b in range(BSZ)],
        axis=0)


def _col(vec_ref):
    return jnp.transpose(jnp.reshape(vec_ref[...], (1, -1)))


def _col4(vec_ref):
    c = _col(vec_ref)
    return jnp.concatenate([c] * BSZ, axis=0)


def _fused_kernel(x0t_ref, b1_hbm, b2_hbm,
                  w_enc0_ref, b_enc0_ref, th_e_ref, th_t_ref,
                  w_enc1_ref, b_enc1_ref, w_enc2_ref, b_enc2_ref,
                  w_c0_ref, w_c1_ref, w_c2_ref, alpha_ref,
                  w_f_ref, b_f_ref,
                  out_ref, stage, b2_s, sems):
    def _issue(i):
        src = b1_hbm if i < NSPLIT else b2_hbm
        c = i % NSPLIT
        pltpu.make_async_copy(src.at[pl.ds(c * RSPLIT, RSPLIT)],
                              stage.at[i % NSTAGE],
                              sems.at[i]).start()

    def _wait(i):
        src = b1_hbm if i < NSPLIT else b2_hbm
        c = i % NSPLIT
        pltpu.make_async_copy(src.at[pl.ds(c * RSPLIT, RSPLIT)],
                              stage.at[i % NSTAGE],
                              sems.at[i]).wait()

    for i in range(NSTAGE):
        _issue(i)

    alpha = alpha_ref[0]
    w_enc0 = w_enc0_ref[...]
    x0h = _swish(jnp.concatenate(
        [_dot(w_enc0, x0t_ref[:, b * S:(b + 1) * S], ((0,), (0,)))
         for b in range(BSZ)], axis=0) + _col4(b_enc0_ref))

    t1 = None
    for i in range(NSPLIT):
        _wait(i)
        p = _dot(x0h[:, i * RSPLIT:(i + 1) * RSPLIT],
                 stage[i % NSTAGE], ((1,), (0,)))
        t1 = p if t1 is None else t1 + p
    x1 = _blockmix(th_e_ref[...], t1)
    w_f = w_f_ref[...]
    b_f = _col(b_f_ref)
    w_c0 = w_c0_ref[...]
    l0_1 = _blockmix(w_c0, x0h)
    acc = [_dot(w_f[0:HID], x0h[b * HID:(b + 1) * HID], ((0,), (0,)))
           for b in range(BSZ)]

    t2 = None
    for i in range(NSPLIT, 2 * NSPLIT):
        c = i % NSPLIT
        _wait(i)
        half = stage[i % NSTAGE]
        b2_s[c * RSPLIT:(c + 1) * RSPLIT, :] = half
        p = _dot(x1[:, c * RSPLIT:(c + 1) * RSPLIT], half, ((1,), (0,)))
        t2 = p if t2 is None else t2 + p
    x2 = _blockmix(th_t_ref[...], t2)
    x2h = _blockmix(w_enc2_ref[...], x2) + _col4(b_enc2_ref)

    B2 = b2_s[...]
    w_c2 = w_c2_ref[...]

    for b in range(BSZ):
        sl = slice(b * HID, (b + 1) * HID)
        x2h_b = x2h[sl]
        up0 = _dot(x2h_b, B2, ((1,), (0,)))
        x0h_1 = _swish(up0 + alpha * (l0_1[sl] - up0))
        x2h_1 = _swish(_dot(w_c2, x2h_b, ((0,), (0,))))
        a = acc[b] + _dot(w_f[HID:2 * HID], x0h_1, ((0,), (0,)))
        l0_2 = _dot(w_c0, x0h_1, ((0,), (0,)))
        up0b = _dot(x2h_1, B2, ((1,), (0,)))
        x0h_2 = _swish(up0b + alpha * (l0_2 - up0b))
        a = a + _dot(w_f[2 * HID:3 * HID], x0h_2, ((0,), (0,)))
        out_ref[b] = _swish(a + b_f)


def kernel(x, pos, batch, triangles, B1, B2, W_enc0, b_enc0, theta_edge,
           theta_tri, W_enc1, b_enc1, W_enc2, b_enc2, W_conv0, W_conv1,
           W_conv2, alpha, W_tproj, b_tproj, W_dec, b_dec):
    vfull = lambda shp: pl.BlockSpec(shp, lambda: (0,) * len(shp))
    hbm = pl.BlockSpec(memory_space=pl.ANY)
    smem1 = pl.BlockSpec(memory_space=pltpu.SMEM)

    x0t = jnp.concatenate([x.T, pos.T], axis=0)
    w_f = jnp.dot(W_tproj, W_dec, precision='highest')
    b_f = jnp.dot(b_tproj, W_dec, precision='highest') + b_dec

    out = pl.pallas_call(
        _fused_kernel,
        in_specs=[
            vfull((5, BSZ * S)),
            hbm, hbm,
            vfull((5, HID)), vfull((HID,)),
            vfull((HID, HID)), vfull((HID, HID)),
            vfull((HID, HID)), vfull((HID,)),
            vfull((HID, HID)), vfull((HID,)),
            vfull((HID, HID)), vfull((HID, HID)), vfull((HID, HID)),
            smem1,
            vfull((HID * TEMPORAL_STEPS, TIME_STEPS)), vfull((TIME_STEPS,)),
        ],
        out_specs=vfull((BSZ, TIME_STEPS, S)),
        out_shape=jax.ShapeDtypeStruct((BSZ, TIME_STEPS, S), jnp.float32),
        scratch_shapes=[
            pltpu.VMEM((NSTAGE, RSPLIT, S), jnp.float32),
            pltpu.VMEM((S, S), jnp.float32),
            pltpu.SemaphoreType.DMA((2 * NSPLIT,)),
        ],
        compiler_params=pltpu.CompilerParams(
            vmem_limit_bytes=110 * 1024 * 1024),
    )(
        x0t, B1, B2,
        W_enc0, b_enc0, theta_edge, theta_tri,
        W_enc1, b_enc1, W_enc2, b_enc2,
        W_conv0, W_conv1, W_conv2, alpha.reshape(1),
        w_f, b_f,
    )
    return out


# speedup vs baseline: 1.0528x; 1.0432x over previous
import jax
import jax.numpy as jnp
from jax.experimental import pallas as pl
from jax.experimental.pallas import tpu as pltpu

S = 2048
HID = 128
BSZ = 4
STACK = BSZ * HID
TIME_STEPS = 10
TEMPORAL_STEPS = 3
NCHUNK = 8
RCHUNK = S // NCHUNK


def _swish(v):
    return v * jax.nn.sigmoid(v)


def _dot(a, b, dims):
    return jax.lax.dot_general(
        a, b, (dims, ((), ())), preferred_element_type=jnp.float32)


def _blockmix(w, xs):
    return jnp.concatenate(
        [_dot(w, xs[b * HID:(b + 1) * HID], ((0,), (0,))) for b in range(BSZ)],
        axis=0)


def _col(vec_ref):
    return jnp.transpose(jnp.reshape(vec_ref[...], (1, -1)))


def _col4(vec_ref):
    c = _col(vec_ref)
    return jnp.concatenate([c] * BSZ, axis=0)


def _fused_kernel(x0t_ref, b1_hbm, b2_hbm,
                  w_enc0_ref, b_enc0_ref, th_e_ref, th_t_ref,
                  w_enc1_ref, b_enc1_ref, w_enc2_ref, b_enc2_ref,
                  w_c0_ref, w_c1_ref, w_c2_ref, alpha_ref,
                  w_f_ref, b_f_ref,
                  out_ref, b1_s, b2_s, sems):
    for c in range(NCHUNK):
        pltpu.make_async_copy(b1_hbm.at[pl.ds(c * RCHUNK, RCHUNK)],
                              b1_s.at[pl.ds(c * RCHUNK, RCHUNK)],
                              sems.at[c]).start()
    for c in range(NCHUNK):
        pltpu.make_async_copy(b2_hbm.at[pl.ds(c * RCHUNK, RCHUNK)],
                              b2_s.at[pl.ds(c * RCHUNK, RCHUNK)],
                              sems.at[NCHUNK + c]).start()

    alpha = alpha_ref[0]
    w_enc0 = w_enc0_ref[...]
    x0h = _swish(jnp.concatenate(
        [_dot(w_enc0, x0t_ref[:, b * S:(b + 1) * S], ((0,), (0,)))
         for b in range(BSZ)], axis=0) + _col4(b_enc0_ref))

    t1 = None
    for c in range(NCHUNK):
        pltpu.make_async_copy(b1_hbm.at[pl.ds(c * RCHUNK, RCHUNK)],
                              b1_s.at[pl.ds(c * RCHUNK, RCHUNK)],
                              sems.at[c]).wait()
        p = _dot(x0h[:, c * RCHUNK:(c + 1) * RCHUNK],
                 b1_s[c * RCHUNK:(c + 1) * RCHUNK], ((1,), (0,)))
        t1 = p if t1 is None else t1 + p
    x1 = _blockmix(th_e_ref[...], t1)
    x1h = _swish(_blockmix(w_enc1_ref[...], x1) + _col4(b_enc1_ref))

    t2 = None
    for c in range(NCHUNK):
        pltpu.make_async_copy(b2_hbm.at[pl.ds(c * RCHUNK, RCHUNK)],
                              b2_s.at[pl.ds(c * RCHUNK, RCHUNK)],
                              sems.at[NCHUNK + c]).wait()
        p = _dot(x1[:, c * RCHUNK:(c + 1) * RCHUNK],
                 b2_s[c * RCHUNK:(c + 1) * RCHUNK], ((1,), (0,)))
        t2 = p if t2 is None else t2 + p
    x2 = _blockmix(th_t_ref[...], t2)
    x2h = _blockmix(w_enc2_ref[...], x2) + _col4(b_enc2_ref)

    B2 = b2_s[...]
    bundled = [x0h]
    for _ in range(TEMPORAL_STEPS - 1):
        x0_lower = _blockmix(w_c0_ref[...], bundled[-1])
        x0_upper = _dot(x2h, B2, ((1,), (0,)))
        x0h_new = _swish(x0_upper + alpha * (x0_lower - x0_upper))
        x1_lower = _blockmix(w_c1_ref[...], x1h)
        x1_upper = _dot(x2h, B2, ((1,), (1,)))
        x1h = _swish(0.5 * (x1_lower + x1_upper))
        x2h = _swish(_blockmix(w_c2_ref[...], x2h))
        bundled.append(x0h_new)

    w_f = w_f_ref[...]
    b_f = _col(b_f_ref)
    for b in range(BSZ):
        acc = None
        for k in range(TEMPORAL_STEPS):
            p = _dot(w_f[k * HID:(k + 1) * HID],
                     bundled[k][b * HID:(b + 1) * HID], ((0,), (0,)))
            acc = p if acc is None else acc + p
        out_ref[b] = _swish(acc + b_f)


def kernel(x, pos, batch, triangles, B1, B2, W_enc0, b_enc0, theta_edge,
           theta_tri, W_enc1, b_enc1, W_enc2, b_enc2, W_conv0, W_conv1,
           W_conv2, alpha, W_tproj, b_tproj, W_dec, b_dec):
    vfull = lambda shp: pl.BlockSpec(shp, lambda: (0,) * len(shp))
    hbm = pl.BlockSpec(memory_space=pl.ANY)
    smem1 = pl.BlockSpec(memory_space=pltpu.SMEM)

    x0t = jnp.concatenate([x.T, pos.T], axis=0)
    w_f = jnp.dot(W_tproj, W_dec, precision='highest')
    b_f = jnp.dot(b_tproj, W_dec, precision='highest') + b_dec

    out = pl.pallas_call(
        _fused_kernel,
        in_specs=[
            vfull((5, BSZ * S)),
            hbm, hbm,
            vfull((5, HID)), vfull((HID,)),
            vfull((HID, HID)), vfull((HID, HID)),
            vfull((HID, HID)), vfull((HID,)),
            vfull((HID, HID)), vfull((HID,)),
            vfull((HID, HID)), vfull((HID, HID)), vfull((HID, HID)),
            smem1,
            vfull((HID * TEMPORAL_STEPS, TIME_STEPS)), vfull((TIME_STEPS,)),
        ],
        out_specs=vfull((BSZ, TIME_STEPS, S)),
        out_shape=jax.ShapeDtypeStruct((BSZ, TIME_STEPS, S), jnp.float32),
        scratch_shapes=[
            pltpu.VMEM((S, S), jnp.float32),
            pltpu.VMEM((S, S), jnp.float32),
            pltpu.SemaphoreType.DMA((2 * NCHUNK,)),
        ],
        compiler_params=pltpu.CompilerParams(
            vmem_limit_bytes=110 * 1024 * 1024),
    )(
        x0t, B1, B2,
        W_enc0, b_enc0, theta_edge, theta_tri,
        W_enc1, b_enc1, W_enc2, b_enc2,
        W_conv0, W_conv1, W_conv2, alpha.reshape(1),
        w_f, b_f,
    )
    return out
